# baseline (speedup 1.0000x reference)
import functools

import jax
import jax.numpy as jnp
from jax import lax
from jax.experimental import pallas as pl
from jax.experimental.pallas import tpu as pltpu
from jax.experimental.pallas import tpu_sc as plsc

_NC = 2
_NS = 16
_NW = _NC * _NS
_CHUNK = 128


def _sc_gather(idx2d, table2, n_chunks):
    m = idx2d.shape[0] * idx2d.shape[1]
    d2 = table2.shape[1]
    b_per_w = n_chunks * _CHUNK
    mesh = plsc.VectorSubcoreMesh(core_axis_name="c", subcore_axis_name="s")

    @functools.partial(
        pl.kernel,
        mesh=mesh,
        out_type=jax.ShapeDtypeStruct((m, d2), jnp.float32),
        scratch_types=[
            pltpu.VMEM((n_chunks, _CHUNK), jnp.int32),
            pltpu.VMEM((b_per_w, d2), jnp.float32),
            pltpu.SemaphoreType.DMA,
        ],
    )
    def gather_kernel(idx_hbm, table_hbm, out_hbm, idx_v, rows_v, sem):
        wid = lax.axis_index("s") * _NC + lax.axis_index("c")
        pltpu.sync_copy(idx_hbm.at[pl.ds(wid * n_chunks, n_chunks)], idx_v)
        copies = []
        for j in range(n_chunks):
            copies.append(
                pltpu.async_copy(
                    table_hbm.at[idx_v.at[j]],
                    rows_v.at[pl.ds(j * _CHUNK, _CHUNK)],
                    sem,
                )
            )
        for c in copies:
            c.wait()
        pltpu.sync_copy(rows_v, out_hbm.at[pl.ds(wid * b_per_w, b_per_w)])

    return gather_kernel(idx2d, table2)


def _tc_select_matmul(ve2, par, w, block_m):
    m, d2 = ve2.shape
    d = d2 // 2
    dm = w.shape[0]

    def mm_body(ve2_ref, par_ref, w_ref, out_ref):
        lo = ve2_ref[:, :d]
        hi = ve2_ref[:, d:]
        ve = jnp.where(par_ref[...] > 0, hi, lo)
        out_ref[...] = lax.dot_general(
            ve,
            w_ref[...],
            (((1,), (1,)), ((), ())),
            preferred_element_type=jnp.float32,
        )

    return pl.pallas_call(
        mm_body,
        grid=(m // block_m,),
        in_specs=[
            pl.BlockSpec((block_m, d2), lambda i: (i, 0)),
            pl.BlockSpec((block_m, 1), lambda i: (i, 0)),
            pl.BlockSpec((dm, d), lambda i: (0, 0)),
        ],
        out_specs=pl.BlockSpec((block_m, dm), lambda i: (i, 0)),
        out_shape=jax.ShapeDtypeStruct((m, dm), jnp.float32),
    )(ve2, par, w)


def kernel(tokens, emb, W):
    b, s = tokens.shape
    m = b * s
    v, d = emb.shape
    dm = W.shape[0]
    n_chunks = m // (_NW * _CHUNK)
    tok = tokens.reshape(m).astype(jnp.int32)
    idx2d = (tok // 2).reshape(_NW * n_chunks, _CHUNK)
    par = (tok % 2).astype(jnp.float32).reshape(m, 1)
    table2 = emb.reshape(v // 2, 2 * d)
    ve2 = _sc_gather(idx2d, table2, n_chunks)
    out = _tc_select_matmul(ve2, par, W, 2048)
    return out.reshape(b, s, dm)


# speedup vs baseline: 2.0568x; 2.0568x over previous
import functools

import jax
import jax.numpy as jnp
from jax import lax
from jax.experimental import pallas as pl
from jax.experimental.pallas import tpu as pltpu
from jax.experimental.pallas import tpu_sc as plsc

_NC = 2
_NS = 16
_NW = _NC * _NS
_CHUNK = 128


def _tc_relayout(embT, block_v):
    d, v = embT.shape
    h = block_v // 2
    n_blocks = -(-v // block_v)

    def body(in_ref, out_ref):
        x = in_ref[...]
        out_ref[:, :d] = jnp.transpose(x[:, :h], (1, 0))
        out_ref[:, d:] = jnp.transpose(x[:, h:], (1, 0))

    return pl.pallas_call(
        body,
        grid=(n_blocks,),
        in_specs=[pl.BlockSpec((d, block_v), lambda i: (0, i))],
        out_specs=pl.BlockSpec((h, 2 * d), lambda i: (i, 0)),
        out_shape=jax.ShapeDtypeStruct((n_blocks * h, 2 * d), jnp.float32),
    )(embT)


def _sc_gather(idx2d, table2, n_chunks):
    m = idx2d.shape[0] * idx2d.shape[1]
    d2 = table2.shape[1]
    b_per_w = n_chunks * _CHUNK
    mesh = plsc.VectorSubcoreMesh(core_axis_name="c", subcore_axis_name="s")

    @functools.partial(
        pl.kernel,
        mesh=mesh,
        out_type=jax.ShapeDtypeStruct((m, d2), jnp.float32),
        scratch_types=[
            pltpu.VMEM((n_chunks, _CHUNK), jnp.int32),
            pltpu.VMEM((b_per_w, d2), jnp.float32),
            pltpu.SemaphoreType.DMA,
        ],
    )
    def gather_kernel(idx_hbm, table_hbm, out_hbm, idx_v, rows_v, sem):
        wid = lax.axis_index("s") * _NC + lax.axis_index("c")
        pltpu.sync_copy(idx_hbm.at[pl.ds(wid * n_chunks, n_chunks)], idx_v)
        copies = []
        for j in range(n_chunks):
            copies.append(
                pltpu.async_copy(
                    table_hbm.at[idx_v.at[j]],
                    rows_v.at[pl.ds(j * _CHUNK, _CHUNK)],
                    sem,
                )
            )
        for c in copies:
            c.wait()
        pltpu.sync_copy(rows_v, out_hbm.at[pl.ds(wid * b_per_w, b_per_w)])

    return gather_kernel(idx2d, table2)


def _tc_select_matmul(ve2, par, w, block_m):
    m, d2 = ve2.shape
    d = d2 // 2
    dm = w.shape[0]

    def mm_body(ve2_ref, par_ref, w_ref, out_ref):
        lo = ve2_ref[:, :d]
        hi = ve2_ref[:, d:]
        ve = jnp.where(par_ref[...] > 0, hi, lo)
        out_ref[...] = lax.dot_general(
            ve,
            w_ref[...],
            (((1,), (1,)), ((), ())),
            preferred_element_type=jnp.float32,
        )

    return pl.pallas_call(
        mm_body,
        grid=(m // block_m,),
        in_specs=[
            pl.BlockSpec((block_m, d2), lambda i: (i, 0)),
            pl.BlockSpec((block_m, 1), lambda i: (i, 0)),
            pl.BlockSpec((dm, d), lambda i: (0, 0)),
        ],
        out_specs=pl.BlockSpec((block_m, dm), lambda i: (i, 0)),
        out_shape=jax.ShapeDtypeStruct((m, dm), jnp.float32),
    )(ve2, par, w)


def kernel(tokens, emb, W):
    b, s = tokens.shape
    m = b * s
    v, d = emb.shape
    dm = W.shape[0]
    n_chunks = m // (_NW * _CHUNK)
    block_v = 8192
    h = block_v // 2
    tok = tokens.reshape(m).astype(jnp.int32)
    idx = (tok // block_v) * h + (tok % h)
    idx2d = idx.reshape(_NW * n_chunks, _CHUNK)
    par = (((tok % block_v) // h)).astype(jnp.float32).reshape(m, 1)
    table2 = _tc_relayout(emb.T, block_v)
    ve2 = _sc_gather(idx2d, table2, n_chunks)
    out = _tc_select_matmul(ve2, par, W, 2048)
    return out.reshape(b, s, dm)
